# baseline (speedup 1.0000x reference)
import functools

import jax
import jax.numpy as jnp
from jax import lax
from jax.experimental import pallas as pl
from jax.experimental.pallas import tpu as pltpu
from jax.experimental.pallas import tpu_sc as plsc

NC = 2
NS = 16
L = 16
NW = NC * NS

BATCH = 4096
SEQ = 200
POS_ROWS = 256
FEAT = 64
BW = BATCH // NW
NPAIR = SEQ // 2
_ABLATE_COMPUTE = False


def _body(tab, tokt, pos, out, idx_v, pos_v, g0, g1, o0, o1,
          gs0, gs1, ws0, ws1):
    wid = lax.axis_index("s") * NC + lax.axis_index("c")
    b0 = pl.multiple_of(wid * BW, BW)
    pltpu.sync_copy(tokt.at[pl.ds(0, SEQ), pl.ds(b0, BW)], idx_v)
    pltpu.sync_copy(pos, pos_v)

    lanes = lax.iota(jnp.int32, L)
    trows = [tg * L + lanes for tg in range(BW // L)]

    def issue_gather(l, buf, sem):
        pltpu.async_copy(tab.at[idx_v.at[l]], buf, sem)

    def wait_gather(l, buf, sem):
        pltpu.make_async_copy(tab.at[idx_v.at[l]], buf, sem).wait()

    def issue_write(l, buf, sem):
        pltpu.async_copy(buf, out.at[l, pl.ds(0, FEAT), pl.ds(b0, BW)], sem)

    def wait_write(buf, sem):
        pltpu.make_async_copy(
            buf, out.at[0, pl.ds(0, FEAT), pl.ds(b0, BW)], sem).wait()

    def compute(l, gbuf, obuf):
        def fgrp(fg, _):
            f0 = fg * L
            pvec = pos_v[l, pl.ds(f0, L)]
            for fi in range(L):
                f = f0 + fi
                pv = jnp.full((L,), pvec[fi], dtype=jnp.float32)
                cols = jnp.full((L,), f, dtype=jnp.int32)
                for tg in range(BW // L):
                    vec = plsc.load_gather(gbuf, [trows[tg], cols]) + pv
                    obuf[f, pl.ds(tg * L, L)] = vec
            return _

        if _ABLATE_COMPUTE:
            return
        lax.fori_loop(0, FEAT // L, fgrp, 0, unroll=False)

    issue_gather(0, g0, gs0)

    def step(i, _):
        l0 = 2 * i
        issue_gather(l0 + 1, g1, gs1)
        wait_gather(l0, g0, gs0)

        @pl.when(i > 0)
        def _w0():
            wait_write(o0, ws0)

        compute(l0, g0, o0)
        issue_write(l0, o0, ws0)

        @pl.when(i < NPAIR - 1)
        def _n0():
            issue_gather(l0 + 2, g0, gs0)

        wait_gather(l0 + 1, g1, gs1)

        @pl.when(i > 0)
        def _w1():
            wait_write(o1, ws1)

        compute(l0 + 1, g1, o1)
        issue_write(l0 + 1, o1, ws1)
        return _

    lax.fori_loop(0, NPAIR, step, 0, unroll=False)
    wait_write(o0, ws0)
    wait_write(o1, ws1)


@jax.jit
def _encode(tab128, tokt, pos2d):
    kern = functools.partial(
        pl.kernel,
        out_type=jax.ShapeDtypeStruct((SEQ, FEAT, BATCH), jnp.float32),
        mesh=plsc.VectorSubcoreMesh(core_axis_name="c", subcore_axis_name="s"),
        scratch_types=[
            pltpu.VMEM((SEQ, 128), jnp.int32),
            pltpu.VMEM((POS_ROWS, FEAT), jnp.float32),
            pltpu.VMEM((128, 128), jnp.float32),
            pltpu.VMEM((128, 128), jnp.float32),
            pltpu.VMEM((FEAT, 128), jnp.float32),
            pltpu.VMEM((FEAT, 128), jnp.float32),
            pltpu.SemaphoreType.DMA,
            pltpu.SemaphoreType.DMA,
            pltpu.SemaphoreType.DMA,
            pltpu.SemaphoreType.DMA,
        ],
        compiler_params=pltpu.CompilerParams(
            use_tc_tiling_on_sc=True, needs_layout_passes=False),
    )(_body)
    return kern(tab128, tokt, pos2d)


def kernel(tokens, embed_table, pos_embedding):
    tab128 = jnp.pad(embed_table, ((0, 0), (0, 128 - FEAT)))
    tokt = tokens.astype(jnp.int32).T
    out_t = _encode(tab128, tokt, pos_embedding[0])
    return jnp.transpose(out_t, (2, 0, 1))


# speedup vs baseline: 1.2471x; 1.2471x over previous
import functools

import jax
import jax.numpy as jnp
from jax import lax
from jax.experimental import pallas as pl
from jax.experimental.pallas import tpu as pltpu
from jax.experimental.pallas import tpu_sc as plsc

NC = 2
NS = 16
L = 16
NW = NC * NS

BATCH = 4096
SEQ = 200
VOCAB = 1000000
FEAT = 64
N = BATCH * SEQ
R_PER_W = N // NW
U = 128
UNITS = R_PER_W // U
NPAIR = UNITS // 2
IDX_BLK = 16
PREP_BLK = 1024


def _prep_body(x_ref, o_ref):
    o_ref[:, :FEAT] = x_ref[...].T


def _prep(tabt):
    return pl.pallas_call(
        _prep_body,
        grid=(pl.cdiv(VOCAB, PREP_BLK),),
        in_specs=[pl.BlockSpec((FEAT, PREP_BLK), lambda i: (0, i))],
        out_specs=pl.BlockSpec((PREP_BLK, 128), lambda i: (i, 0)),
        out_shape=jax.ShapeDtypeStruct((VOCAB, 128), jnp.float32),
    )(tabt)


def _body(tab, toks, pos, out, idx_v, pos_v, b0buf, b1buf, gs0, gs1, ws0, ws1):
    wid = lax.axis_index("s") * NC + lax.axis_index("c")
    row0 = pl.multiple_of(wid * R_PER_W, R_PER_W)
    pltpu.sync_copy(pos, pos_v)

    def stage_idx(blk):
        off = pl.multiple_of(row0 // U + blk * IDX_BLK, 8)
        pltpu.sync_copy(toks.at[pl.ds(off, IDX_BLK)], idx_v)

    def issue_gather(u, buf, sem):
        pltpu.async_copy(tab.at[idx_v.at[lax.rem(u, IDX_BLK)]], buf, sem)

    def wait_gather(u, buf, sem):
        pltpu.make_async_copy(
            tab.at[idx_v.at[lax.rem(u, IDX_BLK)]], buf, sem).wait()

    def issue_write(u, buf, sem):
        pltpu.async_copy(buf, out.at[pl.ds(row0 + u * U, U)], sem)

    def wait_write(buf, sem):
        pltpu.make_async_copy(buf, out.at[pl.ds(row0, U)], sem).wait()

    def add_pos(u, buf):
        def row(r, p):
            for j in range(FEAT // L):
                sl = pl.ds(j * L, L)
                buf[r, sl] = buf[r, sl] + pos_v[p, sl]
            return lax.select(p + 1 == SEQ, 0, p + 1)
        lax.fori_loop(0, U, row, lax.rem(u * U, SEQ), unroll=4)

    stage_idx(0)
    issue_gather(0, b0buf, gs0)

    def step(i, _):
        a = 2 * i

        @pl.when(i > 0)
        def _w1():
            wait_write(b1buf, ws1)

        issue_gather(a + 1, b1buf, gs1)
        wait_gather(a, b0buf, gs0)
        add_pos(a, b0buf)
        issue_write(a, b0buf, ws0)

        @pl.when(lax.rem(a + 2, IDX_BLK) == 0)
        def _stage():
            stage_idx((a + 2) // IDX_BLK)

        wait_write(b0buf, ws0)
        issue_gather(a + 2, b0buf, gs0)
        wait_gather(a + 1, b1buf, gs1)
        add_pos(a + 1, b1buf)
        issue_write(a + 1, b1buf, ws1)
        return _

    lax.fori_loop(0, NPAIR - 1, step, 0, unroll=False)
    a = 2 * (NPAIR - 1)
    wait_write(b1buf, ws1)
    issue_gather(a + 1, b1buf, gs1)
    wait_gather(a, b0buf, gs0)
    add_pos(a, b0buf)
    issue_write(a, b0buf, ws0)
    wait_gather(a + 1, b1buf, gs1)
    add_pos(a + 1, b1buf)
    issue_write(a + 1, b1buf, ws1)
    wait_write(b0buf, ws0)
    wait_write(b1buf, ws1)


@jax.jit
def _encode(tab128, toks2d, pos2d):
    kern = functools.partial(
        pl.kernel,
        out_type=jax.ShapeDtypeStruct((N, 128), jnp.float32),
        mesh=plsc.VectorSubcoreMesh(core_axis_name="c", subcore_axis_name="s"),
        scratch_types=[
            pltpu.VMEM((IDX_BLK, 128), jnp.int32),
            pltpu.VMEM((SEQ, FEAT), jnp.float32),
            pltpu.VMEM((U, 128), jnp.float32),
            pltpu.VMEM((U, 128), jnp.float32),
            pltpu.SemaphoreType.DMA,
            pltpu.SemaphoreType.DMA,
            pltpu.SemaphoreType.DMA,
            pltpu.SemaphoreType.DMA,
        ],
        compiler_params=pltpu.CompilerParams(use_tc_tiling_on_sc=True),
    )(_body)
    return kern(tab128, toks2d, pos2d)


def kernel(tokens, embed_table, pos_embedding):
    tab128 = _prep(embed_table.T)
    toks2d = tokens.astype(jnp.int32).reshape(N // 128, 128)
    out128 = _encode(tab128, toks2d, pos_embedding[0, :SEQ])
    return out128.reshape(BATCH, SEQ, 128)[:, :, :FEAT]
